# Initial kernel scaffold; baseline (speedup 1.0000x reference)
import functools

import jax
import jax.numpy as jnp
from jax import lax
from jax.experimental import pallas as pl
from jax.experimental.pallas import tpu as pltpu
from jax.experimental.pallas import tpu_sc as plsc

NC = 2
NS = 16
LANES = 16


def _matmul(x, W):
    n, d_in = x.shape
    d_out = W.shape[1]
    blk = 1000

    def body(x_ref, w_ref, o_ref):
        o_ref[...] = jnp.dot(
            x_ref[...], w_ref[...],
            preferred_element_type=jnp.float32,
            precision=lax.Precision.HIGHEST,
        )

    return pl.pallas_call(
        body,
        grid=(n // blk,),
        in_specs=[
            pl.BlockSpec((blk, d_in), lambda i: (i, 0)),
            pl.BlockSpec((d_in, d_out), lambda i: (0, 0)),
        ],
        out_specs=pl.BlockSpec((blk, d_out), lambda i: (i, 0)),
        out_shape=jax.ShapeDtypeStruct((n, d_out), jnp.float32),
    )(x, W)


def _scatter_partials(xw, src, dst, ev, zeros):
    n, d = xw.shape
    e = src.shape[0]
    epw = e // (NC * NS)
    chunk = 80
    nchunk = epw // chunk
    rows_per_sub = n // NS

    mesh = plsc.VectorSubcoreMesh(core_axis_name="c", subcore_axis_name="s")

    @functools.partial(
        pl.kernel,
        mesh=mesh,
        out_type=jax.ShapeDtypeStruct((NC * n, d), jnp.float32),
        scratch_types=[
            pltpu.VMEM((chunk,), jnp.int32),
            pltpu.VMEM((chunk,), jnp.int32),
            pltpu.VMEM((chunk,), jnp.float32),
            pltpu.VMEM((chunk, d), jnp.float32),
            pltpu.VMEM_SHARED((n, d), jnp.float32),
            pltpu.SemaphoreType.DMA,
        ],
    )
    def k(xw_hbm, src_hbm, dst_hbm, ev_hbm, z_hbm, out_hbm,
          src_v, dst_v, ev_v, rows_v, acc, sem):
        c = lax.axis_index("c")
        s = lax.axis_index("s")
        my_rows = pl.ds(s * rows_per_sub, rows_per_sub)
        pltpu.sync_copy(z_hbm.at[my_rows], acc.at[my_rows])
        plsc.subcore_barrier()

        wid = s * NC + c
        base = wid * epw

        @pl.loop(0, nchunk)
        def _(ci):
            off = base + ci * chunk
            pltpu.sync_copy(src_hbm.at[pl.ds(off, chunk)], src_v)
            pltpu.sync_copy(dst_hbm.at[pl.ds(off, chunk)], dst_v)
            pltpu.sync_copy(ev_hbm.at[pl.ds(off, chunk)], ev_v)
            pltpu.async_copy(xw_hbm.at[src_v], rows_v, sem).wait()

            @pl.loop(0, chunk)
            def _(i):
                scale = plsc.load_gather(
                    ev_v, [jnp.full((LANES,), i, jnp.int32)])
                for j in range(d // LANES):
                    sl = (i, pl.ds(j * LANES, LANES))
                    rows_v.at[*sl][...] = rows_v.at[*sl][...] * scale

            pltpu.sync_copy(rows_v, acc.at[dst_v], add=True)

        plsc.subcore_barrier()
        pltpu.sync_copy(
            acc.at[my_rows],
            out_hbm.at[pl.ds(c * n + s * rows_per_sub, rows_per_sub)])

    return k(xw, src, dst, ev, zeros)


def _combine_relu(partials):
    _, n, d = partials.shape
    blk = 1000

    def body(p_ref, o_ref):
        o_ref[...] = jnp.maximum(p_ref[0] + p_ref[1], 0.0)

    return pl.pallas_call(
        body,
        grid=(n // blk,),
        in_specs=[pl.BlockSpec((NC, blk, d), lambda i: (0, i, 0))],
        out_specs=pl.BlockSpec((blk, d), lambda i: (i, 0)),
        out_shape=jax.ShapeDtypeStruct((n, d), jnp.float32),
    )(partials)


def kernel(x, edge_index, edge_values, W):
    n, _ = x.shape
    d = W.shape[1]
    xw = _matmul(x, W)
    src = edge_index[1]
    dst = edge_index[0]
    zeros = jnp.zeros((n, d), jnp.float32)
    partials = _scatter_partials(xw, src, dst, edge_values, zeros)
    return _combine_relu(partials.reshape(NC, n, d))


# speedup vs baseline: 3.9416x; 3.9416x over previous
import dataclasses
import functools

import jax
import jax.numpy as jnp
from jax import lax
from jax.experimental import pallas as pl
from jax.experimental.pallas import tpu as pltpu
from jax.experimental.pallas import tpu_sc as plsc

NC = 2
NS = 16
LANES = 16


def _matmul(x, W):
    n, d_in = x.shape
    d_out = W.shape[1]
    blk = 1000

    def body(x_ref, w_ref, o_ref):
        o_ref[...] = jnp.dot(
            x_ref[...], w_ref[...],
            preferred_element_type=jnp.float32,
            precision=lax.Precision.HIGHEST,
        )

    return pl.pallas_call(
        body,
        grid=(n // blk,),
        in_specs=[
            pl.BlockSpec((blk, d_in), lambda i: (i, 0)),
            pl.BlockSpec((d_in, d_out), lambda i: (0, 0)),
        ],
        out_specs=pl.BlockSpec((blk, d_out), lambda i: (i, 0)),
        out_shape=jax.ShapeDtypeStruct((n, d_out), jnp.float32),
    )(x, W)


def _scatter_partials(xw, src, dst, ev, zeros):
    n, d = xw.shape
    e = src.shape[0]
    epw = e // (NC * NS)
    chunk = 80
    nchunk = epw // chunk
    n_pad = zeros.shape[0]
    rows_per_sub = n_pad // NS

    mesh = plsc.VectorSubcoreMesh(core_axis_name="c", subcore_axis_name="s")
    cp = pltpu.CompilerParams()
    if "needs_layout_passes" in pltpu.CompilerParams.__dataclass_fields__:
        cp = dataclasses.replace(cp, needs_layout_passes=False)

    @functools.partial(
        pl.kernel,
        mesh=mesh,
        compiler_params=cp,
        out_type=jax.ShapeDtypeStruct((NC * n_pad, d), jnp.float32),
        scratch_types=[
            pltpu.VMEM((chunk,), jnp.int32),
            pltpu.VMEM((chunk,), jnp.int32),
            pltpu.VMEM((chunk,), jnp.float32),
            pltpu.VMEM((chunk, d), jnp.float32),
            pltpu.VMEM_SHARED((n_pad, d), jnp.float32),
            pltpu.SemaphoreType.DMA,
        ],
    )
    def k(xw_hbm, src_hbm, dst_hbm, ev_hbm, z_hbm, out_hbm,
          src_v, dst_v, ev_v, rows_v, acc, sem):
        c = lax.axis_index("c")
        s = lax.axis_index("s")
        my_rows = pl.ds(s * rows_per_sub, rows_per_sub)
        pltpu.sync_copy(z_hbm.at[my_rows], acc.at[my_rows])
        plsc.subcore_barrier()

        wid = s * NC + c
        base = wid * epw

        @pl.loop(0, nchunk)
        def _(ci):
            off = base + ci * chunk
            pltpu.sync_copy(src_hbm.at[pl.ds(off, chunk)], src_v)
            pltpu.sync_copy(dst_hbm.at[pl.ds(off, chunk)], dst_v)
            pltpu.sync_copy(ev_hbm.at[pl.ds(off, chunk)], ev_v)
            pltpu.async_copy(xw_hbm.at[src_v], rows_v, sem).wait()

            @pl.loop(0, chunk)
            def _(i):
                scale = plsc.load_gather(
                    ev_v, [jnp.full((LANES,), i, jnp.int32)])
                for j in range(d // LANES):
                    sl = (i, pl.ds(j * LANES, LANES))
                    rows_v.at[*sl][...] = rows_v.at[*sl][...] * scale

            pltpu.sync_copy(rows_v, acc.at[dst_v], add=True)

        plsc.subcore_barrier()
        pltpu.sync_copy(
            acc.at[my_rows],
            out_hbm.at[pl.ds(c * n_pad + s * rows_per_sub, rows_per_sub)])

    return k(xw, src, dst, ev, zeros)


def _combine_relu(partials, n):
    d = partials.shape[-1]
    blk = 1000

    def body(p_ref, o_ref):
        o_ref[...] = jnp.maximum(p_ref[0] + p_ref[1], 0.0)

    return pl.pallas_call(
        body,
        grid=(n // blk,),
        in_specs=[pl.BlockSpec((NC, blk, d), lambda i: (0, i, 0))],
        out_specs=pl.BlockSpec((blk, d), lambda i: (i, 0)),
        out_shape=jax.ShapeDtypeStruct((n, d), jnp.float32),
    )(partials)


def kernel(x, edge_index, edge_values, W):
    n, _ = x.shape
    d = W.shape[1]
    xw = _matmul(x, W)
    src = edge_index[1]
    dst = edge_index[0]
    n_pad = ((n + 8 * NS - 1) // (8 * NS)) * (8 * NS)
    zeros = jnp.zeros((n_pad, d), jnp.float32)
    partials = _scatter_partials(xw, src, dst, edge_values, zeros)
    return _combine_relu(partials.reshape(NC, n_pad, d), n)


# speedup vs baseline: 8.1809x; 2.0755x over previous
import dataclasses
import functools

import jax
import jax.numpy as jnp
from jax import lax
from jax.experimental import pallas as pl
from jax.experimental.pallas import tpu as pltpu
from jax.experimental.pallas import tpu_sc as plsc

NC = 2
NS = 16
LANES = 16


def _matmul(x, W):
    n, d_in = x.shape
    d_out = W.shape[1]
    blk = 1000

    def body(x_ref, w_ref, o_ref):
        o_ref[...] = jnp.dot(
            x_ref[...], w_ref[...],
            preferred_element_type=jnp.float32,
            precision=lax.Precision.HIGHEST,
        )

    return pl.pallas_call(
        body,
        grid=(n // blk,),
        in_specs=[
            pl.BlockSpec((blk, d_in), lambda i: (i, 0)),
            pl.BlockSpec((d_in, d_out), lambda i: (0, 0)),
        ],
        out_specs=pl.BlockSpec((blk, d_out), lambda i: (i, 0)),
        out_shape=jax.ShapeDtypeStruct((n, d_out), jnp.float32),
    )(x, W)


def _scatter_partials(xw, src, dst, ev, zeros):
    n, d = xw.shape
    e = src.shape[0]
    nw = NC * NS
    epw = e // nw
    chunk = 80
    nchunk = epw // chunk
    n_pad = zeros.shape[0]
    rows_per_sub = n_pad // NS

    ngroup = 5
    g_e = epw // ngroup
    nchunk_g = g_e // chunk

    src = src.reshape(nw, ngroup, g_e)
    dst = dst.reshape(nw, ngroup, nchunk_g, chunk)
    ev = ev.reshape(nw, ngroup, g_e)

    mesh = plsc.VectorSubcoreMesh(core_axis_name="c", subcore_axis_name="s")
    cp = pltpu.CompilerParams()
    if "needs_layout_passes" in pltpu.CompilerParams.__dataclass_fields__:
        cp = dataclasses.replace(cp, needs_layout_passes=False)

    @functools.partial(
        pl.kernel,
        mesh=mesh,
        compiler_params=cp,
        out_type=jax.ShapeDtypeStruct((NC * n_pad, d), jnp.float32),
        scratch_types=[
            pltpu.VMEM((g_e,), jnp.int32),
            pltpu.VMEM((nchunk_g, chunk), jnp.int32),
            pltpu.VMEM((g_e,), jnp.float32),
            pltpu.VMEM((2, chunk, d), jnp.float32),
            pltpu.VMEM_SHARED((n_pad, d), jnp.float32),
            pltpu.SemaphoreType.DMA,
            pltpu.SemaphoreType.DMA,
        ],
    )
    def k(xw_hbm, src_hbm, dst_hbm, ev_hbm, z_hbm, out_hbm,
          src_v, dst_v, ev_v, rows_v, acc, sem0, sem1):
        c = lax.axis_index("c")
        s = lax.axis_index("s")
        my_rows = pl.ds(s * rows_per_sub, rows_per_sub)
        pltpu.sync_copy(z_hbm.at[my_rows], acc.at[my_rows])

        wid = s * NC + c
        plsc.subcore_barrier()

        sems = [sem0, sem1]

        def gather(ci, b):
            idx = src_v.at[pl.ds(ci * chunk, chunk)]
            return pltpu.make_async_copy(
                xw_hbm.at[idx], rows_v.at[b], sems[b])

        def process(ci, b):
            gather(ci, b).wait()
            rows_b = rows_v.at[b]

            @pl.loop(0, chunk)
            def _(i):
                scale = plsc.load_gather(
                    ev_v, [jnp.full((LANES,), ci * chunk + i, jnp.int32)])
                for j in range(d // LANES):
                    sl = (i, pl.ds(j * LANES, LANES))
                    rows_b.at[*sl][...] = rows_b.at[*sl][...] * scale

            pltpu.sync_copy(rows_b, acc.at[dst_v.at[ci]], add=True)

            @pl.when(ci + 2 < nchunk_g)
            def _():
                gather(ci + 2, b).start()

        @pl.loop(0, ngroup)
        def _(grp):
            pltpu.sync_copy(src_hbm.at[wid, grp], src_v)
            pltpu.sync_copy(dst_hbm.at[wid, grp], dst_v)
            pltpu.sync_copy(ev_hbm.at[wid, grp], ev_v)

            gather(0, 0).start()
            gather(1, 1).start()

            @pl.loop(0, nchunk_g - 1, step=2)
            def _(g):
                process(g, 0)
                process(g + 1, 1)

            process(nchunk_g - 1, 0)

        plsc.subcore_barrier()
        pltpu.sync_copy(
            acc.at[my_rows],
            out_hbm.at[pl.ds(c * n_pad + s * rows_per_sub, rows_per_sub)])

    return k(xw, src, dst, ev, zeros)


def _combine_relu(partials, n):
    d = partials.shape[-1]
    blk = 1000

    def body(p_ref, o_ref):
        o_ref[...] = jnp.maximum(p_ref[0] + p_ref[1], 0.0)

    return pl.pallas_call(
        body,
        grid=(n // blk,),
        in_specs=[pl.BlockSpec((NC, blk, d), lambda i: (0, i, 0))],
        out_specs=pl.BlockSpec((blk, d), lambda i: (i, 0)),
        out_shape=jax.ShapeDtypeStruct((n, d), jnp.float32),
    )(partials)


def kernel(x, edge_index, edge_values, W):
    n, _ = x.shape
    d = W.shape[1]
    xw = _matmul(x, W)
    src = edge_index[1]
    dst = edge_index[0]
    n_pad = ((n + 8 * NS - 1) // (8 * NS)) * (8 * NS)
    zeros = jnp.zeros((n_pad, d), jnp.float32)
    partials = _scatter_partials(xw, src, dst, edge_values, zeros)
    return _combine_relu(partials.reshape(NC, n_pad, d), n)


# speedup vs baseline: 8.4725x; 1.0356x over previous
import dataclasses
import functools

import jax
import jax.numpy as jnp
from jax import lax
from jax.experimental import pallas as pl
from jax.experimental.pallas import tpu as pltpu
from jax.experimental.pallas import tpu_sc as plsc

NC = 2
NS = 16
LANES = 16


def _matmul(x, W):
    n, d_in = x.shape
    d_out = W.shape[1]
    blk = 1000

    def body(x_ref, w_ref, o_ref):
        o_ref[...] = jnp.dot(
            x_ref[...], w_ref[...],
            preferred_element_type=jnp.float32,
            precision=lax.Precision.HIGHEST,
        )

    return pl.pallas_call(
        body,
        grid=(n // blk,),
        in_specs=[
            pl.BlockSpec((blk, d_in), lambda i: (i, 0)),
            pl.BlockSpec((d_in, d_out), lambda i: (0, 0)),
        ],
        out_specs=pl.BlockSpec((blk, d_out), lambda i: (i, 0)),
        out_shape=jax.ShapeDtypeStruct((n, d_out), jnp.float32),
    )(x, W)


def _scatter_partials(xw, src, dst, ev, zeros):
    n, d = xw.shape
    e = src.shape[0]
    nw = NC * NS
    epw = e // nw
    chunk = 80
    nchunk = epw // chunk
    n_pad = zeros.shape[0]
    rows_per_sub = n_pad // NS

    ngroup = 5
    g_e = epw // ngroup
    nchunk_g = g_e // chunk

    src = src.reshape(nw, ngroup, g_e)
    dst = dst.reshape(nw, ngroup, nchunk_g, chunk)
    ev = ev.reshape(nw, ngroup, g_e)

    mesh = plsc.VectorSubcoreMesh(core_axis_name="c", subcore_axis_name="s")
    cp = pltpu.CompilerParams()
    if "needs_layout_passes" in pltpu.CompilerParams.__dataclass_fields__:
        cp = dataclasses.replace(cp, needs_layout_passes=False)

    @functools.partial(
        pl.kernel,
        mesh=mesh,
        compiler_params=cp,
        out_type=jax.ShapeDtypeStruct((NC * n_pad, d), jnp.float32),
        scratch_types=[
            pltpu.VMEM((g_e,), jnp.int32),
            pltpu.VMEM((nchunk_g, chunk), jnp.int32),
            pltpu.VMEM((g_e,), jnp.float32),
            pltpu.VMEM((2, chunk, d), jnp.float32),
            pltpu.VMEM_SHARED((n_pad, d), jnp.float32),
            pltpu.SemaphoreType.DMA,
            pltpu.SemaphoreType.DMA,
        ],
    )
    def k(xw_hbm, src_hbm, dst_hbm, ev_hbm, z_hbm, out_hbm,
          src_v, dst_v, ev_v, rows_v, acc, sem0, sem1):
        c = lax.axis_index("c")
        s = lax.axis_index("s")
        my_rows = pl.ds(s * rows_per_sub, rows_per_sub)
        pltpu.sync_copy(z_hbm.at[my_rows], acc.at[my_rows])

        wid = s * NC + c
        plsc.subcore_barrier()

        sems = [sem0, sem1]

        def gather(ci, b):
            idx = src_v.at[pl.ds(ci * chunk, chunk)]
            return pltpu.make_async_copy(
                xw_hbm.at[idx], rows_v.at[b], sems[b])

        def process(ci, b):
            gather(ci, b).wait()
            rows_b = rows_v.at[b]

            @pl.loop(0, chunk, step=4)
            def _(i):
                for t in range(4):
                    scale = plsc.load_gather(
                        ev_v,
                        [jnp.full((LANES,), ci * chunk + i + t, jnp.int32)])
                    for j in range(d // LANES):
                        sl = (i + t, pl.ds(j * LANES, LANES))
                        rows_b.at[*sl][...] = rows_b.at[*sl][...] * scale

            pltpu.sync_copy(rows_b, acc.at[dst_v.at[ci]], add=True)

            @pl.when(ci + 2 < nchunk_g)
            def _():
                gather(ci + 2, b).start()

        @pl.loop(0, ngroup)
        def _(grp):
            pltpu.sync_copy(src_hbm.at[wid, grp], src_v)
            pltpu.sync_copy(dst_hbm.at[wid, grp], dst_v)
            pltpu.sync_copy(ev_hbm.at[wid, grp], ev_v)

            gather(0, 0).start()
            gather(1, 1).start()

            @pl.loop(0, nchunk_g - 1, step=2)
            def _(g):
                process(g, 0)
                process(g + 1, 1)

            process(nchunk_g - 1, 0)

        plsc.subcore_barrier()
        pltpu.sync_copy(
            acc.at[my_rows],
            out_hbm.at[pl.ds(c * n_pad + s * rows_per_sub, rows_per_sub)])

    return k(xw, src, dst, ev, zeros)


def _combine_relu(partials, n):
    d = partials.shape[-1]
    blk = 1000

    def body(p_ref, o_ref):
        o_ref[...] = jnp.maximum(p_ref[0] + p_ref[1], 0.0)

    return pl.pallas_call(
        body,
        grid=(n // blk,),
        in_specs=[pl.BlockSpec((NC, blk, d), lambda i: (0, i, 0))],
        out_specs=pl.BlockSpec((blk, d), lambda i: (i, 0)),
        out_shape=jax.ShapeDtypeStruct((n, d), jnp.float32),
    )(partials)


def kernel(x, edge_index, edge_values, W):
    n, _ = x.shape
    d = W.shape[1]
    xw = _matmul(x, W)
    src = edge_index[1]
    dst = edge_index[0]
    n_pad = ((n + 8 * NS - 1) // (8 * NS)) * (8 * NS)
    zeros = jnp.zeros((n_pad, d), jnp.float32)
    partials = _scatter_partials(xw, src, dst, edge_values, zeros)
    return _combine_relu(partials.reshape(NC, n_pad, d), n)
